# baseline (speedup 1.0000x reference)
import functools

import jax
import jax.numpy as jnp
from jax import lax
from jax.experimental import pallas as pl
from jax.experimental.pallas import tpu as pltpu

_LANE = 128
_SUBLANE = 8
_TARGET_BLOCK_BYTES = 4 * 1024 * 1024
_CHUNK_BYTES = 1 * 1024 * 1024
_VMEM_LIMIT = 48 * 1024 * 1024


def _flat_reduce_kernel(p1_ref, p2_ref, out_ref, acc_ref, *,
                        steps: int, chunk: int, n_chunks: int,
                        rows: int, block_rows: int, need_mask: bool,
                        inv_scale: float):
    i = pl.program_id(1)

    @pl.when(i == 0)
    def _():
        acc_ref[...] = jnp.zeros_like(acc_ref)

    row0 = (pl.program_id(0) * steps + i) * block_rows

    def accum(k, carry):
        start = pl.multiple_of(k * chunk, _SUBLANE)
        d = p1_ref[pl.ds(start, chunk), :] - p2_ref[pl.ds(start, chunk), :]
        if need_mask:
            rid = row0 + start + lax.broadcasted_iota(jnp.int32, (chunk, 1), 0)
            d = jnp.where(rid < rows, d, 0.0)
        acc_ref[...] += d * d
        return carry

    if n_chunks == 1:
        accum(0, 0)
    else:
        lax.fori_loop(0, n_chunks, accum, 0, unroll=(n_chunks <= 8))

    @pl.when(i == steps - 1)
    def _():
        out_ref[0, 0, 0] = jnp.sum(acc_ref[...]) * inv_scale


def _pick_lanes(n: int) -> int:
    for lanes in (1024, 512, 256, 128):
        if n % lanes == 0 and (n // lanes) % _SUBLANE == 0:
            return lanes
    return 0


def kernel(pred1: jax.Array, pred2: jax.Array):
    assert pred1.shape == pred2.shape and pred1.ndim == 2
    bs, num_classes = pred1.shape
    n = bs * num_classes
    inv_scale = 1.0 / float(num_classes) if num_classes else 0.0

    lanes = _pick_lanes(n) if n else 0
    if lanes == 0:
        d = pred1.astype(jnp.float32) - pred2.astype(jnp.float32)
        return jnp.sum(d * d) / jnp.float32(num_classes), bs

    rows = n // lanes
    p1 = pred1.reshape(rows, lanes)
    p2 = pred2.reshape(rows, lanes)

    itemsize = jnp.dtype(p1.dtype).itemsize
    row_bytes = lanes * itemsize
    target = max(_SUBLANE,
                 min(rows, _TARGET_BLOCK_BYTES // row_bytes)
                 // _SUBLANE * _SUBLANE)
    block_rows = 0
    for br in range(target, max(_SUBLANE, target // 2) - 1, -_SUBLANE):
        if rows % br == 0:
            block_rows = br
            break
    chunk_target = max(_SUBLANE,
                       min(block_rows or target, _CHUNK_BYTES // (lanes * 4))
                       // _SUBLANE * _SUBLANE)
    if block_rows:
        chunk = next((c for c in range(chunk_target, _SUBLANE - 1, -_SUBLANE)
                      if block_rows % c == 0), block_rows)
    else:
        chunk = chunk_target
        block_rows = chunk * max(1, target // chunk)

    nblocks = pl.cdiv(rows, block_rows)
    ncores = 2 if nblocks >= 2 else 1
    steps = pl.cdiv(nblocks, ncores)
    need_mask = ncores * steps * block_rows > rows

    in_map = lambda c, i: (jnp.minimum(c * steps + i, nblocks - 1), 0)
    in_spec = pl.BlockSpec((block_rows, lanes), in_map)

    partials = pl.pallas_call(
        functools.partial(
            _flat_reduce_kernel, steps=steps, chunk=chunk,
            n_chunks=block_rows // chunk, rows=rows, block_rows=block_rows,
            need_mask=need_mask, inv_scale=inv_scale),
        out_shape=jax.ShapeDtypeStruct((ncores, 1, 1), jnp.float32),
        grid=(ncores, steps),
        in_specs=[in_spec, in_spec],
        out_specs=pl.BlockSpec((1, 1, 1), lambda c, i: (c, 0, 0),
                               memory_space=pltpu.SMEM),
        scratch_shapes=[pltpu.VMEM((chunk, lanes), jnp.float32)],
        compiler_params=pltpu.CompilerParams(
            dimension_semantics=("parallel", "arbitrary"),
            vmem_limit_bytes=_VMEM_LIMIT,
        ),
        cost_estimate=pl.CostEstimate(
            flops=3 * n, transcendentals=0,
            bytes_accessed=2 * n * itemsize + ncores * 4),
    )(p1, p2)

    loss = jnp.sum(partials)
    return loss, bs


# speedup vs baseline: 1.7883x; 1.7883x over previous
import functools

import jax
import jax.numpy as jnp
from jax import lax
from jax.experimental import pallas as pl
from jax.experimental.pallas import tpu as pltpu

_SUBLANE = 8
_TARGET_BLOCK_BYTES = 8 * 1024 * 1024
_CHUNK_BYTES = 1 * 1024 * 1024
_VMEM_LIMIT = 56 * 1024 * 1024


def _sq_diff_kernel(p1_ref, p2_ref, out_ref, acc_ref, *,
                    steps: int, chunk: int, n_chunks: int,
                    rows: int, block_rows: int, need_mask: bool,
                    inv_scale: float):
    i = pl.program_id(1)

    @pl.when(i == 0)
    def _():
        acc_ref[...] = jnp.zeros_like(acc_ref)

    row0 = (pl.program_id(0) * steps + i) * block_rows

    def accum(k, carry):
        start = pl.multiple_of(k * chunk, _SUBLANE)
        d = (p1_ref[pl.ds(start, chunk), :].astype(jnp.float32)
             - p2_ref[pl.ds(start, chunk), :].astype(jnp.float32))
        if need_mask:
            rid = row0 + start + lax.broadcasted_iota(jnp.int32, (chunk, 1), 0)
            d = jnp.where(rid < rows, d, 0.0)
        acc_ref[...] += d * d
        return carry

    if n_chunks == 1:
        accum(0, 0)
    else:
        lax.fori_loop(0, n_chunks, accum, 0, unroll=(n_chunks <= 8))

    @pl.when(i == steps - 1)
    def _():
        out_ref[0, 0, 0] = jnp.sum(acc_ref[...]) * inv_scale


def kernel(pred1: jax.Array, pred2: jax.Array):
    assert pred1.shape == pred2.shape and pred1.ndim == 2
    bs, num_classes = pred1.shape
    n = bs * num_classes
    if n == 0:
        return jnp.float32(0.0), bs
    inv_scale = 1.0 / float(num_classes)

    itemsize = jnp.dtype(pred1.dtype).itemsize
    c_pad = max(1, -(-num_classes // 128)) * 128
    row_bytes = c_pad * itemsize
    target = max(_SUBLANE,
                 min(bs, _TARGET_BLOCK_BYTES // row_bytes)
                 // _SUBLANE * _SUBLANE)
    block_rows = 0
    for br in range(target, max(_SUBLANE, target // 2) - 1, -_SUBLANE):
        if bs % br == 0:
            block_rows = br
            break
    chunk_target = max(_SUBLANE,
                       min(block_rows or target, _CHUNK_BYTES // (c_pad * 4))
                       // _SUBLANE * _SUBLANE)
    if block_rows:
        chunk = next((c for c in range(chunk_target, _SUBLANE - 1, -_SUBLANE)
                      if block_rows % c == 0), block_rows)
    else:
        chunk = chunk_target
        block_rows = chunk * max(1, target // chunk)

    nblocks = pl.cdiv(bs, block_rows)
    ncores = 2 if nblocks >= 2 else 1
    steps = pl.cdiv(nblocks, ncores)
    need_mask = ncores * steps * block_rows > bs

    in_map = lambda c, i: (jnp.minimum(c * steps + i, nblocks - 1), 0)
    in_spec = pl.BlockSpec((block_rows, num_classes), in_map)

    partials = pl.pallas_call(
        functools.partial(
            _sq_diff_kernel, steps=steps, chunk=chunk,
            n_chunks=block_rows // chunk, rows=bs, block_rows=block_rows,
            need_mask=need_mask, inv_scale=inv_scale),
        out_shape=jax.ShapeDtypeStruct((ncores, 1, 1), jnp.float32),
        grid=(ncores, steps),
        in_specs=[in_spec, in_spec],
        out_specs=pl.BlockSpec((1, 1, 1), lambda c, i: (c, 0, 0),
                               memory_space=pltpu.SMEM),
        scratch_shapes=[pltpu.VMEM((chunk, num_classes), jnp.float32)],
        compiler_params=pltpu.CompilerParams(
            dimension_semantics=("parallel", "arbitrary"),
            vmem_limit_bytes=_VMEM_LIMIT,
        ),
        cost_estimate=pl.CostEstimate(
            flops=3 * n, transcendentals=0,
            bytes_accessed=2 * n * itemsize + ncores * 4),
    )(pred1, pred2)

    loss = jnp.sum(partials)
    return loss, bs


# speedup vs baseline: 1.8137x; 1.0142x over previous
import functools

import jax
import jax.numpy as jnp
from jax import lax
from jax.experimental import pallas as pl
from jax.experimental.pallas import tpu as pltpu

_SUBLANE = 8
_TARGET_BLOCK_BYTES = 8 * 1024 * 1024
_CHUNK_BYTES = 1 * 1024 * 1024
_VMEM_LIMIT = 56 * 1024 * 1024


def _sq_diff_kernel(p1_ref, p2_ref, out_ref, acc_ref, *,
                    steps: int, chunk: int, n_chunks: int,
                    rows: int, block_rows: int, need_mask: bool,
                    inv_scale: float):
    i = pl.program_id(1)

    @pl.when(i == 0)
    def _():
        acc_ref[...] = jnp.zeros_like(acc_ref)

    row0 = (pl.program_id(0) * steps + i) * block_rows

    def accum(k, carry):
        start = pl.multiple_of(k * chunk, _SUBLANE)
        d = (p1_ref[pl.ds(start, chunk), :].astype(jnp.float32)
             - p2_ref[pl.ds(start, chunk), :].astype(jnp.float32))
        if need_mask:
            rid = row0 + start + lax.broadcasted_iota(jnp.int32, (chunk, 1), 0)
            d = jnp.where(rid < rows, d, 0.0)
        acc_ref[...] += d * d
        return carry

    if n_chunks == 1:
        accum(0, 0)
    else:
        lax.fori_loop(0, n_chunks, accum, 0, unroll=(n_chunks <= 8))

    @pl.when(i == steps - 1)
    def _():
        out_ref[0, 0, 0] = jnp.sum(acc_ref[...]) * inv_scale


def kernel(pred1: jax.Array, pred2: jax.Array):
    assert pred1.shape == pred2.shape and pred1.ndim == 2
    bs, num_classes = pred1.shape
    n = bs * num_classes
    if n == 0:
        return jnp.float32(0.0), bs
    inv_scale = 1.0 / float(num_classes)

    itemsize = jnp.dtype(pred1.dtype).itemsize
    c_pad = max(1, -(-num_classes // 128)) * 128
    row_bytes = c_pad * itemsize
    target = max(_SUBLANE,
                 min(bs, _TARGET_BLOCK_BYTES // row_bytes)
                 // _SUBLANE * _SUBLANE)
    block_rows = 0
    for br in range(target, max(_SUBLANE, target // 2) - 1, -_SUBLANE):
        if bs % br == 0:
            block_rows = br
            break
    chunk_target = max(_SUBLANE,
                       min(block_rows or target, _CHUNK_BYTES // (c_pad * 4))
                       // _SUBLANE * _SUBLANE)
    if block_rows:
        chunk = next((c for c in range(chunk_target, _SUBLANE - 1, -_SUBLANE)
                      if block_rows % c == 0), block_rows)
    else:
        chunk = chunk_target
        block_rows = chunk * max(1, target // chunk)

    nblocks = pl.cdiv(bs, block_rows)
    ncores = 1
    steps = pl.cdiv(nblocks, ncores)
    need_mask = ncores * steps * block_rows > bs

    in_map = lambda c, i: (jnp.minimum(c * steps + i, nblocks - 1), 0)
    in_spec = pl.BlockSpec((block_rows, num_classes), in_map)

    partials = pl.pallas_call(
        functools.partial(
            _sq_diff_kernel, steps=steps, chunk=chunk,
            n_chunks=block_rows // chunk, rows=bs, block_rows=block_rows,
            need_mask=need_mask, inv_scale=inv_scale),
        out_shape=jax.ShapeDtypeStruct((ncores, 1, 1), jnp.float32),
        grid=(ncores, steps),
        in_specs=[in_spec, in_spec],
        out_specs=pl.BlockSpec((1, 1, 1), lambda c, i: (c, 0, 0),
                               memory_space=pltpu.SMEM),
        scratch_shapes=[pltpu.VMEM((chunk, num_classes), jnp.float32)],
        compiler_params=pltpu.CompilerParams(
            dimension_semantics=("parallel", "arbitrary"),
            vmem_limit_bytes=_VMEM_LIMIT,
        ),
        cost_estimate=pl.CostEstimate(
            flops=3 * n, transcendentals=0,
            bytes_accessed=2 * n * itemsize + ncores * 4),
    )(pred1, pred2)

    loss = jnp.sum(partials)
    return loss, bs


# speedup vs baseline: 1.8251x; 1.0063x over previous
import functools

import jax
import jax.numpy as jnp
from jax import lax
from jax.experimental import pallas as pl
from jax.experimental.pallas import tpu as pltpu

_SUBLANE = 8
_STRIPES = 4
_TARGET_BLOCK_BYTES = 2 * 1024 * 1024
_CHUNK_BYTES = 1 * 1024 * 1024
_VMEM_LIMIT = 56 * 1024 * 1024


def _sq_diff_kernel(*refs, stripes: int, steps: int, chunk: int,
                    n_chunks: int, rows: int, block_rows: int,
                    need_mask: bool, inv_scale: float):
    p1_refs = refs[:stripes]
    p2_refs = refs[stripes:2 * stripes]
    out_ref = refs[2 * stripes]
    acc_ref = refs[2 * stripes + 1]
    i = pl.program_id(0)

    @pl.when(i == 0)
    def _():
        acc_ref[...] = jnp.zeros_like(acc_ref)

    for j in range(stripes):
        row0 = (j * steps + i) * block_rows

        def accum(k, carry, p1_ref=p1_refs[j], p2_ref=p2_refs[j], row0=row0):
            start = pl.multiple_of(k * chunk, _SUBLANE)
            d = (p1_ref[pl.ds(start, chunk), :].astype(jnp.float32)
                 - p2_ref[pl.ds(start, chunk), :].astype(jnp.float32))
            if need_mask:
                rid = (row0 + start
                       + lax.broadcasted_iota(jnp.int32, (chunk, 1), 0))
                d = jnp.where(rid < rows, d, 0.0)
            acc_ref[...] += d * d
            return carry

        if n_chunks == 1:
            accum(0, 0)
        else:
            lax.fori_loop(0, n_chunks, accum, 0, unroll=(n_chunks <= 8))

    @pl.when(i == steps - 1)
    def _():
        out_ref[0, 0, 0] = jnp.sum(acc_ref[...]) * inv_scale


def kernel(pred1: jax.Array, pred2: jax.Array):
    assert pred1.shape == pred2.shape and pred1.ndim == 2
    bs, num_classes = pred1.shape
    n = bs * num_classes
    if n == 0:
        return jnp.float32(0.0), bs
    inv_scale = 1.0 / float(num_classes)

    itemsize = jnp.dtype(pred1.dtype).itemsize
    c_pad = max(1, -(-num_classes // 128)) * 128
    row_bytes = c_pad * itemsize
    target = max(_SUBLANE,
                 min(bs, _TARGET_BLOCK_BYTES // row_bytes)
                 // _SUBLANE * _SUBLANE)
    block_rows = 0
    for br in range(target, max(_SUBLANE, target // 2) - 1, -_SUBLANE):
        if bs % br == 0:
            block_rows = br
            break
    chunk_target = max(_SUBLANE,
                       min(block_rows or target, _CHUNK_BYTES // (c_pad * 4))
                       // _SUBLANE * _SUBLANE)
    if block_rows:
        chunk = next((c for c in range(chunk_target, _SUBLANE - 1, -_SUBLANE)
                      if block_rows % c == 0), block_rows)
    else:
        chunk = chunk_target
        block_rows = chunk * max(1, target // chunk)

    nblocks = pl.cdiv(bs, block_rows)
    stripes = min(_STRIPES, nblocks)
    steps = pl.cdiv(nblocks, stripes)
    need_mask = stripes * steps * block_rows > bs

    def stripe_spec(j):
        return pl.BlockSpec(
            (block_rows, num_classes),
            lambda i, j=j: (jnp.minimum(j * steps + i, nblocks - 1), 0))

    in_specs = ([stripe_spec(j) for j in range(stripes)]
                + [stripe_spec(j) for j in range(stripes)])

    partials = pl.pallas_call(
        functools.partial(
            _sq_diff_kernel, stripes=stripes, steps=steps, chunk=chunk,
            n_chunks=block_rows // chunk, rows=bs, block_rows=block_rows,
            need_mask=need_mask, inv_scale=inv_scale),
        out_shape=jax.ShapeDtypeStruct((1, 1, 1), jnp.float32),
        grid=(steps,),
        in_specs=in_specs,
        out_specs=pl.BlockSpec((1, 1, 1), lambda i: (0, 0, 0),
                               memory_space=pltpu.SMEM),
        scratch_shapes=[pltpu.VMEM((chunk, num_classes), jnp.float32)],
        compiler_params=pltpu.CompilerParams(
            dimension_semantics=("arbitrary",),
            vmem_limit_bytes=_VMEM_LIMIT,
        ),
        cost_estimate=pl.CostEstimate(
            flops=3 * n, transcendentals=0,
            bytes_accessed=2 * n * itemsize + 4),
    )(*([pred1] * stripes + [pred2] * stripes))

    loss = jnp.sum(partials)
    return loss, bs
